# Initial kernel scaffold; baseline (speedup 1.0000x reference)
import jax
import jax.numpy as jnp
from jax.experimental import pallas as pl


def kernel(ids, ori_weight, think_weight):
    raise NotImplementedError("write your pallas kernel here")


# speedup vs baseline: 5.7767x; 5.7767x over previous
import functools

import jax
import jax.numpy as jnp
from jax import lax
from jax.experimental import pallas as pl
from jax.experimental.pallas import tpu as pltpu
from jax.experimental.pallas import tpu_sc as plsc

_ORI_V = 100000
_THINK_V = 100000
_EMBED = 64
_N_IDS = 4096 * 200
_NC = 2
_NS = 16
_NW = _NC * _NS
_PER_W = _N_IDS // _NW
_BLK = 6400
_NBLK = _PER_W // _BLK
_TILE = 128
_LIST = _BLK + _TILE
_LANES = 16


def _make_kernel():
    mesh = plsc.VectorSubcoreMesh(core_axis_name="c", subcore_axis_name="s")

    @functools.partial(
        pl.kernel,
        mesh=mesh,
        out_type=jax.ShapeDtypeStruct((_N_IDS, _EMBED), jnp.float32),
        scratch_types=[
            pltpu.VMEM((_BLK,), jnp.int32),
            pltpu.VMEM((_LIST,), jnp.int32),
            pltpu.VMEM((_LIST,), jnp.int32),
            pltpu.VMEM((_LIST,), jnp.int32),
            pltpu.VMEM((_LIST,), jnp.int32),
            pltpu.VMEM((_TILE, _EMBED), jnp.float32),
            pltpu.VMEM((_TILE,), jnp.int32),
            pltpu.SemaphoreType.DMA,
        ],
        compiler_params=pltpu.CompilerParams(
            needs_layout_passes=False, use_tc_tiling_on_sc=False),
    )
    def combined(ids_hbm, ori_hbm, think_hbm, out_hbm,
                 ids_v, la_idx, la_pos, lb_idx, lb_pos, rows_v, pos_v, sem):
        wid = lax.axis_index("s") * _NC + lax.axis_index("c")
        wbase = wid * _PER_W
        iota = lax.iota(jnp.int32, _LANES)

        def do_block(blk, carry):
            base = wbase + blk * _BLK
            pltpu.sync_copy(ids_hbm.at[pl.ds(base, _BLK)], ids_v)

            def compact(i, c):
                ca, cb = c
                v = ids_v[pl.ds(i * _LANES, _LANES)]
                m = v < _ORI_V
                pos = (base + i * _LANES) + iota
                plsc.store_compressed(la_idx.at[pl.ds(ca, _LANES)], v, mask=m)
                plsc.store_compressed(la_pos.at[pl.ds(ca, _LANES)], pos,
                                      mask=m)
                nm = jnp.logical_not(m)
                plsc.store_compressed(lb_idx.at[pl.ds(cb, _LANES)],
                                      v - _ORI_V, mask=nm)
                plsc.store_compressed(lb_pos.at[pl.ds(cb, _LANES)], pos,
                                      mask=nm)
                na = jnp.sum(m.astype(jnp.int32))
                return ca + na, cb + (_LANES - na)

            ca, cb = lax.fori_loop(
                0, _BLK // _LANES, compact,
                (jnp.int32(0), jnp.int32(0)))

            def pad(lst_i, lst_p, cnt):
                last = jnp.full((_LANES,), jnp.maximum(cnt - 1, 0), jnp.int32)
                li = plsc.load_gather(lst_i, [last])
                lp = plsc.load_gather(lst_p, [last])
                for j in range(_TILE // _LANES):
                    lst_i[pl.ds(cnt + j * _LANES, _LANES)] = li
                    lst_p[pl.ds(cnt + j * _LANES, _LANES)] = lp

            pad(la_idx, la_pos, ca)
            pad(lb_idx, lb_pos, cb)

            def run_tiles(lst_i, lst_p, cnt, table):
                ntiles = (cnt + _TILE - 1) // _TILE

                def one(j, c2):
                    off = j * _TILE
                    for q in range(_TILE // _LANES):
                        pos_v[pl.ds(q * _LANES, _LANES)] = (
                            lst_p[pl.ds(off + q * _LANES, _LANES)])
                    pltpu.async_copy(
                        table.at[lst_i.at[pl.ds(off, _TILE)]],
                        rows_v, sem).wait()
                    pltpu.async_copy(rows_v, out_hbm.at[pos_v], sem).wait()
                    return c2

                lax.fori_loop(0, ntiles, one, 0)

            run_tiles(la_idx, la_pos, ca, ori_hbm)
            run_tiles(lb_idx, lb_pos, cb, think_hbm)
            return carry

        lax.fori_loop(0, _NBLK, do_block, 0)

    return combined


_COMBINED = _make_kernel()


def kernel(ids, ori_weight, think_weight):
    flat_ids = ids.reshape(-1).astype(jnp.int32)
    out = _COMBINED(flat_ids, ori_weight, think_weight)
    return out.reshape(ids.shape + (_EMBED,))


# speedup vs baseline: 6.8448x; 1.1849x over previous
import functools

import jax
import jax.numpy as jnp
from jax import lax
from jax.experimental import pallas as pl
from jax.experimental.pallas import tpu as pltpu
from jax.experimental.pallas import tpu_sc as plsc

_ORI_V = 100000
_THINK_V = 100000
_EMBED = 64
_N_IDS = 4096 * 200
_NC = 2
_NS = 16
_NW = _NC * _NS
_PER_W = _N_IDS // _NW
_BLK = 6400
_NBLK = _PER_W // _BLK
_TILE = 128
_GRP = 4
_LIST = _BLK + _TILE
_LANES = 16
_TILE_BYTES = _TILE * _EMBED * 4


def _make_kernel():
    mesh = plsc.VectorSubcoreMesh(core_axis_name="c", subcore_axis_name="s")

    @functools.partial(
        pl.kernel,
        mesh=mesh,
        out_type=jax.ShapeDtypeStruct((_N_IDS, _EMBED), jnp.float32),
        scratch_types=[
            pltpu.VMEM((_BLK,), jnp.int32),
            pltpu.VMEM((_LIST,), jnp.int32),
            pltpu.VMEM((_LIST,), jnp.int32),
            pltpu.VMEM((_LIST,), jnp.int32),
            pltpu.VMEM((_LIST,), jnp.int32),
            pltpu.VMEM((2, _GRP, _TILE, _EMBED), jnp.float32),
            pltpu.VMEM((2, _GRP, _TILE), jnp.int32),
            pltpu.SemaphoreType.DMA,
            pltpu.SemaphoreType.DMA,
            pltpu.SemaphoreType.DMA,
            pltpu.SemaphoreType.DMA,
        ],
        compiler_params=pltpu.CompilerParams(
            needs_layout_passes=False, use_tc_tiling_on_sc=False),
    )
    def combined(ids_hbm, ori_hbm, think_hbm, out_hbm,
                 ids_v, la_idx, la_pos, lb_idx, lb_pos, rows_v, pos_v,
                 gsem0, gsem1, ssem0, ssem1):
        gsems = (gsem0, gsem1)
        ssems = (ssem0, ssem1)
        wid = lax.axis_index("s") * _NC + lax.axis_index("c")
        wbase = wid * _PER_W
        iota = lax.iota(jnp.int32, _LANES)

        def do_block(blk, carry):
            base = wbase + blk * _BLK
            pltpu.sync_copy(ids_hbm.at[pl.ds(base, _BLK)], ids_v)

            def compact(i, c):
                ca, cb = c
                v = ids_v[pl.ds(i * _LANES, _LANES)]
                m = v < _ORI_V
                pos = (base + i * _LANES) + iota
                plsc.store_compressed(la_idx.at[pl.ds(ca, _LANES)], v, mask=m)
                plsc.store_compressed(la_pos.at[pl.ds(ca, _LANES)], pos,
                                      mask=m)
                nm = jnp.logical_not(m)
                plsc.store_compressed(lb_idx.at[pl.ds(cb, _LANES)],
                                      v - _ORI_V, mask=nm)
                plsc.store_compressed(lb_pos.at[pl.ds(cb, _LANES)], pos,
                                      mask=nm)
                na = jnp.sum(m.astype(jnp.int32))
                return ca + na, cb + (_LANES - na)

            ca, cb = lax.fori_loop(
                0, _BLK // _LANES, compact,
                (jnp.int32(0), jnp.int32(0)))

            def pad(lst_i, lst_p, cnt):
                last = jnp.full((_LANES,), jnp.maximum(cnt - 1, 0), jnp.int32)
                li = plsc.load_gather(lst_i, [last])
                lp = plsc.load_gather(lst_p, [last])
                for j in range(_TILE // _LANES):
                    lst_i[pl.ds(cnt + j * _LANES, _LANES)] = li
                    lst_p[pl.ds(cnt + j * _LANES, _LANES)] = lp

            pad(la_idx, la_pos, ca)
            pad(lb_idx, lb_pos, cb)

            def run_list(lst_i, lst_p, cnt, table):
                ntiles = (cnt + _TILE - 1) // _TILE

                def n_of(g):
                    return jnp.where(
                        g >= 0,
                        jnp.clip(ntiles - g * _GRP, 0, _GRP),
                        0)

                def wait_gathers(pq, n):
                    def w(i, c2):
                        pltpu.make_async_copy(
                            table.at[lst_i.at[pl.ds(0, _TILE)]],
                            rows_v.at[pq, i], gsems[pq]).wait()
                        return c2
                    lax.fori_loop(0, n, w, 0)

                def wait_scatters(pq, n):
                    def w(i, c2):
                        pltpu.make_async_copy(
                            rows_v.at[pq, i],
                            out_hbm.at[pos_v.at[pq, i]], ssems[pq]).wait()
                        return c2
                    lax.fori_loop(0, n, w, 0)

                def do_group(g, p):
                    q = 1 - p
                    wait_scatters(p, n_of(g - 2))

                    def fire_g(i, c2):
                        t = g * _GRP + i
                        off = t * _TILE
                        prow = pos_v.at[p, i]
                        for qq in range(_TILE // _LANES):
                            prow[pl.ds(qq * _LANES, _LANES)] = (
                                lst_p[pl.ds(off + qq * _LANES, _LANES)])
                        pltpu.async_copy(
                            table.at[lst_i.at[pl.ds(off, _TILE)]],
                            rows_v.at[p, i], gsems[p])
                        return c2

                    lax.fori_loop(0, n_of(g), fire_g, 0)

                    ngm1 = n_of(g - 1)
                    wait_gathers(q, ngm1)

                    def fire_s(i, c2):
                        pltpu.async_copy(
                            rows_v.at[q, i],
                            out_hbm.at[pos_v.at[q, i]], ssems[q])
                        return c2

                    lax.fori_loop(0, ngm1, fire_s, 0)

                ngroups = (ntiles + _GRP - 1) // _GRP
                npairs = ngroups // 2 + 1

                def pair(gp, c2):
                    do_group(2 * gp, 0)
                    do_group(2 * gp + 1, 1)
                    return c2

                lax.fori_loop(0, npairs, pair, 0)
                wait_scatters(0, n_of(2 * npairs - 2))

            run_list(la_idx, la_pos, ca, ori_hbm)
            run_list(lb_idx, lb_pos, cb, think_hbm)
            return carry

        lax.fori_loop(0, _NBLK, do_block, 0)

    return combined


_COMBINED = _make_kernel()


def kernel(ids, ori_weight, think_weight):
    flat_ids = ids.reshape(-1).astype(jnp.int32)
    out = _COMBINED(flat_ids, ori_weight, think_weight)
    return out.reshape(ids.shape + (_EMBED,))
